# baseline (speedup 1.0000x reference)
import functools

import jax
import jax.numpy as jnp
from jax import lax
from jax.experimental import pallas as pl
from jax.experimental.pallas import tpu as pltpu
from jax.experimental.pallas import tpu_sc as plsc

_NUM_CLASSES = 1000
_N0 = 4096
_N1 = 26
_CHUNK = 2
_ROWS = _CHUNK * _N1
_NC = 2
_NS = 16
_NW = _NC * _NS
_SLABS = _N0 // _NW
_NCHUNK = _SLABS // _CHUNK
_LANES = 16


def _scatter_val(zbuf, idx_ref, local_off, value):
    val = jnp.full((_LANES,), value, jnp.int32)
    for t in range(4):
        p0 = t * _LANES
        pv = lax.iota(jnp.int32, _LANES) + p0
        ia = jnp.where(pv >= _N1, 1, 0).astype(jnp.int32)
        ij = pv - ia * _N1
        ic = idx_ref[pl.ds(local_off + p0, _LANES)]
        if p0 + _LANES <= _ROWS:
            plsc.store_scatter(zbuf, [ia, ij, ic], val)
        else:
            mask = lax.iota(jnp.int32, _LANES) < (_ROWS - p0)
            plsc.store_scatter(zbuf, [ia, ij, ic], val, mask=mask)


@functools.partial(
    pl.kernel,
    out_type=jax.ShapeDtypeStruct((_N0, _N1, _NUM_CLASSES), jnp.int32),
    mesh=plsc.VectorSubcoreMesh(core_axis_name="c", subcore_axis_name="s"),
    compiler_params=pltpu.CompilerParams(use_tc_tiling_on_sc=False, needs_layout_passes=False),
    scratch_types=[
        pltpu.VMEM((_SLABS * _N1,), jnp.int32),
        pltpu.VMEM((_CHUNK, _N1, _NUM_CLASSES), jnp.int32),
        pltpu.VMEM((_CHUNK, _N1, _NUM_CLASSES), jnp.int32),
        pltpu.SemaphoreType.DMA,
        pltpu.SemaphoreType.DMA,
    ],
)
def _one_hot_sc(x_ref, zero_ref, out_ref, idxbuf, zbuf0, zbuf1, sem0, sem1):
    cid = lax.axis_index("c")
    sid = lax.axis_index("s")
    wid = sid * _NC + cid
    a0 = wid * _SLABS

    pltpu.sync_copy(x_ref.at[pl.ds(a0 * _N1, _SLABS * _N1)], idxbuf)
    pltpu.sync_copy(zero_ref, zbuf0)
    pltpu.sync_copy(zero_ref, zbuf1)

    bufs = (zbuf0, zbuf1)
    sems = (sem0, sem1)

    def fire(cc, zbuf, sem):
        _scatter_val(zbuf, idxbuf, cc * _ROWS, 1)
        pltpu.async_copy(zbuf, out_ref.at[pl.ds(a0 + cc * _CHUNK, _CHUNK)], sem)

    def drain_undo(cc, zbuf, sem):
        pltpu.make_async_copy(
            zbuf, out_ref.at[pl.ds(a0, _CHUNK)], sem
        ).wait()
        _scatter_val(zbuf, idxbuf, cc * _ROWS, 0)

    fire(0, bufs[0], sems[0])
    fire(1, bufs[1], sems[1])

    def loop_body(g, carry):
        drain_undo(2 * g - 2, bufs[0], sems[0])
        fire(2 * g, bufs[0], sems[0])
        drain_undo(2 * g - 1, bufs[1], sems[1])
        fire(2 * g + 1, bufs[1], sems[1])
        return carry

    lax.fori_loop(1, _NCHUNK // 2, loop_body, 0)

    for b in range(2):
        pltpu.make_async_copy(
            bufs[b], out_ref.at[pl.ds(a0, _CHUNK)], sems[b]
        ).wait()


def kernel(x1):
    x_flat = x1.reshape(_N0 * _N1).astype(jnp.int32)
    zeros = jnp.zeros((_CHUNK, _N1, _NUM_CLASSES), jnp.int32)
    return _one_hot_sc(x_flat, zeros)


# speedup vs baseline: 1.9634x; 1.9634x over previous
import functools

import jax
import jax.numpy as jnp
from jax import lax
from jax.experimental import pallas as pl
from jax.experimental.pallas import tpu as pltpu
from jax.experimental.pallas import tpu_sc as plsc

_NUM_CLASSES = 1000
_N0 = 4096
_N1 = 26
_NC = 2
_NS = 16
_NW = _NC * _NS
_SLABS = _N0 // _NW
_NCHUNK = _SLABS


@functools.partial(
    pl.kernel,
    out_type=jax.ShapeDtypeStruct((_N0, _N1, _NUM_CLASSES), jnp.int32),
    mesh=plsc.VectorSubcoreMesh(core_axis_name="c", subcore_axis_name="s"),
    scratch_types=[
        pltpu.VMEM((_SLABS, _N1), jnp.int32),
        pltpu.VMEM((1, _N1, _NUM_CLASSES), jnp.int32),
        pltpu.VMEM((1, _N1, _NUM_CLASSES), jnp.int32),
        pltpu.SemaphoreType.DMA,
        pltpu.SemaphoreType.DMA,
    ],
)
def _one_hot_sc(x_ref, zero_ref, out_ref, idxbuf, zbuf0, zbuf1, sem0, sem1):
    cid = lax.axis_index("c")
    sid = lax.axis_index("s")
    wid = sid * _NC + cid
    a0 = wid * _SLABS

    pltpu.sync_copy(x_ref.at[pl.ds(a0, _SLABS)], idxbuf)
    pltpu.sync_copy(zero_ref, zbuf0)
    pltpu.sync_copy(zero_ref, zbuf1)

    bufs = (zbuf0, zbuf1)
    sems = (sem0, sem1)

    lanes = lax.iota(jnp.int32, 16)

    def set_ones(cc, zbuf, value):
        v0 = idxbuf[cc, pl.ds(0, 16)]
        v1 = idxbuf[cc, pl.ds(_N1 - 16, 16)]
        for j in range(_N1):
            c = v0[j] if j < 16 else v1[j - (_N1 - 16)]
            cb = pl.multiple_of((c >> 4) << 4, 16)
            vec = jnp.where(lanes == c - cb, value, 0).astype(jnp.int32)
            zbuf[0, j, pl.ds(cb, 16)] = vec

    def fire(cc, zbuf, sem):
        set_ones(cc, zbuf, 1)
        pltpu.async_copy(zbuf, out_ref.at[pl.ds(a0 + cc, 1)], sem)

    def drain_undo(cc, zbuf, sem):
        pltpu.make_async_copy(
            zbuf, out_ref.at[pl.ds(a0, 1)], sem
        ).wait()
        set_ones(cc, zbuf, 0)

    fire(0, bufs[0], sems[0])
    fire(1, bufs[1], sems[1])

    def loop_body(g, carry):
        drain_undo(2 * g - 2, bufs[0], sems[0])
        fire(2 * g, bufs[0], sems[0])
        drain_undo(2 * g - 1, bufs[1], sems[1])
        fire(2 * g + 1, bufs[1], sems[1])
        return carry

    lax.fori_loop(1, _NCHUNK // 2, loop_body, 0)

    for b in range(2):
        pltpu.make_async_copy(
            bufs[b], out_ref.at[pl.ds(a0, 1)], sems[b]
        ).wait()


def kernel(x1):
    zeros = jnp.zeros((1, _N1, _NUM_CLASSES), jnp.int32)
    return _one_hot_sc(x1.astype(jnp.int32), zeros)


# speedup vs baseline: 6.3489x; 3.2336x over previous
import functools

import jax
import jax.numpy as jnp
from jax import lax
from jax.experimental import pallas as pl
from jax.experimental.pallas import tpu as pltpu
from jax.experimental.pallas import tpu_sc as plsc

_NUM_CLASSES = 1000
_N0 = 4096
_N1 = 26
_NC = 2
_NS = 16
_NW = _NC * _NS
_AW = _N0 // _NW
_LANES = 16
_GROUPS = _AW // _LANES


@functools.partial(
    pl.kernel,
    out_type=jax.ShapeDtypeStruct((_N1, _NUM_CLASSES, _N0), jnp.int32),
    mesh=plsc.VectorSubcoreMesh(core_axis_name="c", subcore_axis_name="s"),
    scratch_types=[
        pltpu.VMEM((_AW,), jnp.int32),
        pltpu.VMEM((_NUM_CLASSES, _AW), jnp.int32),
    ],
)
def _one_hot_sc(xt_ref, zero_ref, out_ref, colbuf, zbuf):
    cid = lax.axis_index("c")
    sid = lax.axis_index("s")
    wid = sid * _NC + cid
    a0 = wid * _AW

    pltpu.sync_copy(zero_ref, zbuf)

    lanes = lax.iota(jnp.int32, _LANES)

    def set_ones(clear):
        vs = [colbuf[pl.ds(g * _LANES, _LANES)] for g in range(_GROUPS)]
        for a in range(_AW):
            g, l = a // _LANES, a % _LANES
            c = vs[g][l]
            ab = g * _LANES
            vec = jnp.where(lanes == l, 1, 0).astype(jnp.int32)
            r = zbuf[c, pl.ds(ab, _LANES)]
            if clear:
                zbuf[c, pl.ds(ab, _LANES)] = r & (1 - vec)
            else:
                zbuf[c, pl.ds(ab, _LANES)] = r | vec

    def loop_body(j, carry):
        pltpu.sync_copy(xt_ref.at[pl.ds(j * _N0 + a0, _AW)], colbuf)
        set_ones(clear=False)
        pltpu.sync_copy(zbuf, out_ref.at[j, :, pl.ds(a0, _AW)])
        set_ones(clear=True)
        return carry

    lax.fori_loop(0, _N1, loop_body, 0)


def kernel(x1):
    xt = x1.astype(jnp.int32).T.reshape(_N1 * _N0)
    zeros = jnp.zeros((_NUM_CLASSES, _AW), jnp.int32)
    out = _one_hot_sc(xt, zeros)
    return out.transpose(2, 0, 1)
